# baseline (speedup 1.0000x reference)
import functools

import jax
import jax.numpy as jnp
from jax import lax
from jax.experimental import pallas as pl
from jax.experimental.pallas import tpu as pltpu
from jax.experimental.pallas import tpu_sc as plsc

N = 10000
E = 320000
D_IN = 128
D_H = 128

NC = 2
NS = 16
NW = NC * NS
CHUNK = 128
KC = 80
KCC = 160
EPAD = NW * KC * CHUNK
NACC = 16 * 632
DDEG = 16

_MESH = plsc.VectorSubcoreMesh(core_axis_name="c", subcore_axis_name="s")


def _fill_2d(ref, rows, cols, value):
    vec = jnp.full((16,), value, dtype=jnp.float32)

    def body(r, _):
        for k in range(cols // 16):
            ref[r, pl.ds(k * 16, 16)] = vec
        return 0

    lax.fori_loop(0, rows, body, 0)


def _msg_ring(feat_hbm, acc, src_v, dst_v, bufs, gsems, ssems, n_chunks):
    nb = len(bufs)
    for b in range(nb):
        pltpu.async_copy(feat_hbm.at[src_v.at[b]], bufs[b], gsems[b])

    def body(t, _):
        for b in range(nb):
            j = nb * t + b
            pltpu.make_async_copy(feat_hbm.at[src_v.at[j]], bufs[b], gsems[b]).wait()
            pltpu.async_copy(bufs[b], acc.at[dst_v.at[j]], ssems[b], add=True)

            @pl.when(j + nb < n_chunks)
            def _():
                pltpu.make_async_copy(bufs[b], acc.at[dst_v.at[j]], ssems[b]).wait()
                pltpu.async_copy(feat_hbm.at[src_v.at[j + nb]], bufs[b], gsems[b])

        return 0

    lax.fori_loop(0, n_chunks // nb, body, 0)
    for b in range(nb):
        j = n_chunks - nb + b
        pltpu.make_async_copy(bufs[b], acc.at[dst_v.at[j]], ssems[b]).wait()


def _zero_acc_slice(zeros_ref, acc, base):
    for k in range(4):
        pltpu.sync_copy(zeros_ref, acc.at[pl.ds(base + k * 128, 128), :])
    pltpu.sync_copy(zeros_ref.at[pl.ds(0, 120), :],
                    acc.at[pl.ds(base + 512, 120), :])


def _write_acc_slice(acc, out_slice, base):
    for k in range(4):
        pltpu.sync_copy(acc.at[pl.ds(base + k * 128, 128), :],
                        out_slice.at[pl.ds(base + k * 128, 128), :])
    pltpu.sync_copy(acc.at[pl.ds(base + 512, 120), :],
                    out_slice.at[pl.ds(base + 512, 120), :])


NCH = 2500
B16 = NCH // 16
X16 = NCH % 16
B32 = NCH // 32
X32 = NCH % 32


def _load_idx_16(edge_hbm, which, s, idx_v):
    begin = s * B16 + jnp.minimum(s, X16)
    pltpu.sync_copy(edge_hbm.at[which, pl.ds(begin, B16), :],
                    idx_v.at[pl.ds(0, B16), :])

    @pl.when(s < X16)
    def _():
        pltpu.sync_copy(edge_hbm.at[which, pl.ds(begin + B16, 1), :],
                        idx_v.at[pl.ds(B16, 1), :])


def _load_idx_32(edge_hbm, which, wid, idx_v):
    begin = wid * B32 + jnp.minimum(wid, X32)
    pltpu.sync_copy(edge_hbm.at[which, pl.ds(begin, B32), :],
                    idx_v.at[pl.ds(0, B32), :])

    @pl.when(wid < X32)
    def _():
        pltpu.sync_copy(edge_hbm.at[which, pl.ds(begin + B32, 1), :],
                        idx_v.at[pl.ds(B32, 1), :])


def _msg_tail(feat_hbm, acc, src_v, dst_v, buf, gsem, j):
    pltpu.async_copy(feat_hbm.at[src_v.at[j]], buf, gsem).wait()
    pltpu.sync_copy(buf, acc.at[dst_v.at[j]], add=True)


@functools.partial(
    pl.kernel,
    out_type=jax.ShapeDtypeStruct((NC, NACC, DDEG), jnp.float32),
    mesh=_MESH,
    compiler_params=pltpu.CompilerParams(use_tc_tiling_on_sc=False),
    scratch_types=[
        pltpu.VMEM((B32 + 1, CHUNK), jnp.int32),
        pltpu.VMEM((CHUNK, DDEG), jnp.float32),
        pltpu.VMEM((CHUNK, DDEG), jnp.float32),
        [pltpu.SemaphoreType.DMA for _ in range(4)],
        pltpu.VMEM_SHARED((NACC, DDEG), jnp.float32),
    ],
)
def _deg_kernel(edge_hbm, out_hbm, dst_v, ones_v, zeros_v, ssems, acc):
    c = lax.axis_index("c")
    s = lax.axis_index("s")
    wid = s * NC + c
    base = s * 632

    _fill_2d(ones_v, CHUNK, DDEG, 1.0)
    _fill_2d(zeros_v, CHUNK, DDEG, 0.0)
    _zero_acc_slice(zeros_v, acc, base)
    plsc.subcore_barrier()

    _load_idx_32(edge_hbm, 1, wid, dst_v)

    for b in range(4):
        pltpu.async_copy(ones_v, acc.at[dst_v.at[b]], ssems[b], add=True)

    def body(t, _):
        for b in range(4):
            j = 4 * t + b
            pltpu.make_async_copy(ones_v, acc.at[dst_v.at[j]], ssems[b]).wait()

            @pl.when(j + 4 < 76)
            def _():
                pltpu.async_copy(ones_v, acc.at[dst_v.at[j + 4]], ssems[b], add=True)

        return 0

    lax.fori_loop(0, 76 // 4, body, 0)
    for j in (76, 77):
        pltpu.sync_copy(ones_v, acc.at[dst_v.at[j]], add=True)

    @pl.when(wid < X32)
    def _():
        pltpu.sync_copy(ones_v, acc.at[dst_v.at[B32]], add=True)

    plsc.subcore_barrier()
    _write_acc_slice(acc, out_hbm.at[c], base)


_DH2 = 64


@functools.partial(
    pl.kernel,
    out_type=jax.ShapeDtypeStruct((NC, NACC, _DH2), jnp.float32),
    mesh=_MESH,
    compiler_params=pltpu.CompilerParams(use_tc_tiling_on_sc=False),
    scratch_types=[
        pltpu.VMEM((B16 + 1, CHUNK), jnp.int32),
        pltpu.VMEM((B16 + 1, CHUNK), jnp.int32),
        [pltpu.VMEM((CHUNK, _DH2), jnp.float32) for _ in range(4)],
        [pltpu.SemaphoreType.DMA for _ in range(4)],
        [pltpu.SemaphoreType.DMA for _ in range(4)],
        pltpu.VMEM_SHARED((NACC, _DH2), jnp.float32),
    ],
)
def _msg128_kernel(edge_hbm, f0_hbm, f1_hbm, out_hbm,
                   src_v, dst_v, bufs, gsems, ssems, acc):
    c = lax.axis_index("c")
    s = lax.axis_index("s")
    base = s * 632

    _fill_2d(bufs[0], CHUNK, _DH2, 0.0)
    _zero_acc_slice(bufs[0], acc, base)
    plsc.subcore_barrier()

    _load_idx_16(edge_hbm, 0, s, src_v)
    _load_idx_16(edge_hbm, 1, s, dst_v)

    def run(feat_hbm):
        _msg_ring(feat_hbm, acc, src_v, dst_v, bufs, gsems, ssems, B16)

        @pl.when(s < X16)
        def _():
            _msg_tail(feat_hbm, acc, src_v, dst_v, bufs[0], gsems[0], B16)

    @pl.when(c == 0)
    def _():
        run(f0_hbm)

    @pl.when(c == 1)
    def _():
        run(f1_hbm)

    plsc.subcore_barrier()
    _write_acc_slice(acc, out_hbm.at[c], base)


_D2 = 16


@functools.partial(
    pl.kernel,
    out_type=jax.ShapeDtypeStruct((NC, NACC, _D2), jnp.float32),
    mesh=_MESH,
    compiler_params=pltpu.CompilerParams(use_tc_tiling_on_sc=False),
    scratch_types=[
        pltpu.VMEM((B32 + 1, CHUNK), jnp.int32),
        pltpu.VMEM((B32 + 1, CHUNK), jnp.int32),
        [pltpu.VMEM((CHUNK, _D2), jnp.float32) for _ in range(4)],
        [pltpu.SemaphoreType.DMA for _ in range(4)],
        [pltpu.SemaphoreType.DMA for _ in range(4)],
        pltpu.VMEM_SHARED((NACC, _D2), jnp.float32),
        pltpu.VMEM_SHARED((N, _D2), jnp.float32),
    ],
)
def _msg16_kernel(edge_hbm, feat_hbm, out_hbm,
                  src_v, dst_v, bufs, gsems, ssems, acc, feat_spm):
    c = lax.axis_index("c")
    s = lax.axis_index("s")
    wid = s * NC + c
    base = s * 632

    _fill_2d(bufs[0], CHUNK, _D2, 0.0)
    _zero_acc_slice(bufs[0], acc, base)
    for k in range(4):
        off = base + k * 128

        @pl.when(off + 128 <= N)
        def _(off=off):
            pltpu.sync_copy(feat_hbm.at[pl.ds(off, 128), :],
                            feat_spm.at[pl.ds(off, 128), :])

    @pl.when(base + 512 + 120 <= N)
    def _():
        pltpu.sync_copy(feat_hbm.at[pl.ds(base + 512, 120), :],
                        feat_spm.at[pl.ds(base + 512, 120), :])

    @pl.when(s == 15)
    def _():
        pltpu.sync_copy(feat_hbm.at[pl.ds(9984, 16), :],
                        feat_spm.at[pl.ds(9984, 16), :])

    plsc.subcore_barrier()

    _load_idx_32(edge_hbm, 0, wid, src_v)
    _load_idx_32(edge_hbm, 1, wid, dst_v)

    _msg_ring(feat_spm, acc, src_v, dst_v, bufs, gsems, ssems, 76)
    for j in (76, 77):
        _msg_tail(feat_spm, acc, src_v, dst_v, bufs[0], gsems[0], j)

    @pl.when(wid < X32)
    def _():
        _msg_tail(feat_spm, acc, src_v, dst_v, bufs[0], gsems[0], B32)

    plsc.subcore_barrier()
    _write_acc_slice(acc, out_hbm.at[c], base)


_RB = 10000
_GRID = N // _RB


def _dinv_of(degp_ref):
    deg = degp_ref[0, :, 0:1] + degp_ref[1, :, 0:1] + 1.0
    return lax.rsqrt(deg)


def _tc_pre_body(x_ref, w1_ref, degp_ref, f0_ref, f1_ref):
    dinv = _dinv_of(degp_ref)
    xw = dinv * jnp.dot(x_ref[...], w1_ref[...], preferred_element_type=jnp.float32)
    f0_ref[...] = xw[:, :_DH2]
    f1_ref[...] = xw[:, _DH2:]


def _tc_mid_body(a0_ref, a1_ref, f0_ref, f1_ref, degp_ref, b1_ref, w2_ref, out_ref):
    dinv = _dinv_of(degp_ref)
    h0 = jnp.tanh(dinv * (a0_ref[0] + f0_ref[...]) + b1_ref[:_DH2])
    h1 = jnp.tanh(dinv * (a1_ref[0] + f1_ref[...]) + b1_ref[_DH2:])
    hw2 = dinv * (
        jnp.dot(h0, w2_ref[:_DH2], preferred_element_type=jnp.float32)
        + jnp.dot(h1, w2_ref[_DH2:], preferred_element_type=jnp.float32))
    out_ref[...] = jnp.concatenate(
        [hw2, jnp.zeros((hw2.shape[0], _D2 - hw2.shape[1]), jnp.float32)], axis=1)


def _tc_post_body(c0_ref, c1_ref, hw_ref, degp_ref, b2_ref, wc_ref, bc_ref, out_ref):
    dinv = _dinv_of(degp_ref)
    nd = b2_ref.shape[0]
    emb = jnp.tanh(
        dinv * (c0_ref[0, :, :nd] + c1_ref[0, :, :nd] + hw_ref[:, :nd]) + b2_ref[...])
    out_ref[...] = jax.nn.sigmoid(
        jnp.dot(emb, wc_ref[...], preferred_element_type=jnp.float32) + bc_ref[...])


def _row_block(d):
    return pl.BlockSpec((_RB, d), lambda i: (i, 0))


def _acc_block(part, d):
    return pl.BlockSpec((1, _RB, d), lambda i, p=part: (p, i, 0))


def _deg_block():
    return pl.BlockSpec((2, _RB, DDEG), lambda i: (0, i, 0))


def _full(shape):
    return pl.BlockSpec(shape, lambda i: tuple(0 for _ in shape))


def kernel(x, edge_index, W1, b1, W2, b2, Wc, bc):
    f32 = jnp.float32
    edge3d = edge_index.reshape(2, NCH, CHUNK)

    degp = _deg_kernel(edge3d)

    f0, f1 = pl.pallas_call(
        _tc_pre_body,
        grid=(_GRID,),
        in_specs=[_row_block(D_IN), _full((D_IN, D_H)), _deg_block()],
        out_specs=[_row_block(_DH2), _row_block(_DH2)],
        out_shape=[jax.ShapeDtypeStruct((N, _DH2), f32),
                   jax.ShapeDtypeStruct((N, _DH2), f32)],
    )(x, W1, degp)

    acc1 = _msg128_kernel(edge3d, f0, f1)

    hw2s = pl.pallas_call(
        _tc_mid_body,
        grid=(_GRID,),
        in_specs=[_acc_block(0, _DH2), _acc_block(1, _DH2),
                  _row_block(_DH2), _row_block(_DH2),
                  _deg_block(), _full((D_H,)), _full((D_H, W2.shape[1]))],
        out_specs=_row_block(_D2),
        out_shape=jax.ShapeDtypeStruct((N, _D2), f32),
    )(acc1, acc1, f0, f1, degp, b1, W2)

    acc2 = _msg16_kernel(edge3d, hw2s)

    out = pl.pallas_call(
        _tc_post_body,
        grid=(_GRID,),
        in_specs=[_acc_block(0, _D2), _acc_block(1, _D2), _row_block(_D2),
                  _deg_block(), _full((W2.shape[1],)),
                  _full((Wc.shape[0], Wc.shape[1])), _full((Wc.shape[1],))],
        out_specs=_row_block(Wc.shape[1]),
        out_shape=jax.ShapeDtypeStruct((N, Wc.shape[1]), f32),
    )(acc2, acc2, hw2s, degp, b2, Wc, bc)

    return out


# speedup vs baseline: 1.0110x; 1.0110x over previous
import functools

import jax
import jax.numpy as jnp
from jax import lax
from jax.experimental import pallas as pl
from jax.experimental.pallas import tpu as pltpu
from jax.experimental.pallas import tpu_sc as plsc

N = 10000
E = 320000
D_IN = 128
D_H = 128

NC = 2
NS = 16
NW = NC * NS
CHUNK = 128
KC = 80
KCC = 160
EPAD = NW * KC * CHUNK
NACC = 16 * 632
DDEG = 16

_MESH = plsc.VectorSubcoreMesh(core_axis_name="c", subcore_axis_name="s")


def _fill_2d(ref, rows, cols, value):
    vec = jnp.full((16,), value, dtype=jnp.float32)

    def body(r, _):
        for k in range(cols // 16):
            ref[r, pl.ds(k * 16, 16)] = vec
        return 0

    lax.fori_loop(0, rows, body, 0)


def _msg_ring(feat_hbm, acc, src_v, dst_v, bufs, gsems, ssems, n_chunks):
    nb = len(bufs)
    for b in range(nb):
        pltpu.async_copy(feat_hbm.at[src_v.at[b]], bufs[b], gsems[b])

    def body(t, _):
        for b in range(nb):
            j = nb * t + b
            pltpu.make_async_copy(feat_hbm.at[src_v.at[j]], bufs[b], gsems[b]).wait()
            pltpu.async_copy(bufs[b], acc.at[dst_v.at[j]], ssems[b], add=True)

            @pl.when(j + nb < n_chunks)
            def _():
                pltpu.make_async_copy(bufs[b], acc.at[dst_v.at[j]], ssems[b]).wait()
                pltpu.async_copy(feat_hbm.at[src_v.at[j + nb]], bufs[b], gsems[b])

        return 0

    lax.fori_loop(0, n_chunks // nb, body, 0)
    for b in range(nb):
        j = n_chunks - nb + b
        pltpu.make_async_copy(bufs[b], acc.at[dst_v.at[j]], ssems[b]).wait()


def _zero_acc_slice(zeros_ref, acc, base):
    for k in range(4):
        pltpu.sync_copy(zeros_ref, acc.at[pl.ds(base + k * 128, 128), :])
    pltpu.sync_copy(zeros_ref.at[pl.ds(0, 120), :],
                    acc.at[pl.ds(base + 512, 120), :])


def _write_acc_slice(acc, out_slice, base):
    for k in range(4):
        pltpu.sync_copy(acc.at[pl.ds(base + k * 128, 128), :],
                        out_slice.at[pl.ds(base + k * 128, 128), :])
    pltpu.sync_copy(acc.at[pl.ds(base + 512, 120), :],
                    out_slice.at[pl.ds(base + 512, 120), :])


NCH = 2500
B16 = NCH // 16
X16 = NCH % 16
B32 = NCH // 32
X32 = NCH % 32


def _load_idx_16(edge_hbm, which, s, idx_v):
    begin = s * B16 + jnp.minimum(s, X16)
    pltpu.sync_copy(edge_hbm.at[which, pl.ds(begin, B16), :],
                    idx_v.at[pl.ds(0, B16), :])

    @pl.when(s < X16)
    def _():
        pltpu.sync_copy(edge_hbm.at[which, pl.ds(begin + B16, 1), :],
                        idx_v.at[pl.ds(B16, 1), :])


def _load_idx_32(edge_hbm, which, wid, idx_v):
    begin = wid * B32 + jnp.minimum(wid, X32)
    pltpu.sync_copy(edge_hbm.at[which, pl.ds(begin, B32), :],
                    idx_v.at[pl.ds(0, B32), :])

    @pl.when(wid < X32)
    def _():
        pltpu.sync_copy(edge_hbm.at[which, pl.ds(begin + B32, 1), :],
                        idx_v.at[pl.ds(B32, 1), :])


def _msg_tail(feat_hbm, acc, src_v, dst_v, buf, gsem, j):
    pltpu.async_copy(feat_hbm.at[src_v.at[j]], buf, gsem).wait()
    pltpu.sync_copy(buf, acc.at[dst_v.at[j]], add=True)


@functools.partial(
    pl.kernel,
    out_type=jax.ShapeDtypeStruct((NC, NACC, DDEG), jnp.float32),
    mesh=_MESH,
    compiler_params=pltpu.CompilerParams(use_tc_tiling_on_sc=False),
    scratch_types=[
        pltpu.VMEM((B32 + 1, CHUNK), jnp.int32),
        pltpu.VMEM((CHUNK, DDEG), jnp.float32),
        pltpu.VMEM((CHUNK, DDEG), jnp.float32),
        [pltpu.SemaphoreType.DMA for _ in range(4)],
        pltpu.VMEM_SHARED((NACC, DDEG), jnp.float32),
    ],
)
def _deg_kernel(edge_hbm, out_hbm, dst_v, ones_v, zeros_v, ssems, acc):
    c = lax.axis_index("c")
    s = lax.axis_index("s")
    wid = s * NC + c
    base = s * 632

    _fill_2d(ones_v, CHUNK, DDEG, 1.0)
    _fill_2d(zeros_v, CHUNK, DDEG, 0.0)
    _zero_acc_slice(zeros_v, acc, base)
    plsc.subcore_barrier()

    _load_idx_32(edge_hbm, 1, wid, dst_v)

    for b in range(4):
        pltpu.async_copy(ones_v, acc.at[dst_v.at[b]], ssems[b], add=True)

    def body(t, _):
        for b in range(4):
            j = 4 * t + b
            pltpu.make_async_copy(ones_v, acc.at[dst_v.at[j]], ssems[b]).wait()

            @pl.when(j + 4 < 76)
            def _():
                pltpu.async_copy(ones_v, acc.at[dst_v.at[j + 4]], ssems[b], add=True)

        return 0

    lax.fori_loop(0, 76 // 4, body, 0)
    for j in (76, 77):
        pltpu.sync_copy(ones_v, acc.at[dst_v.at[j]], add=True)

    @pl.when(wid < X32)
    def _():
        pltpu.sync_copy(ones_v, acc.at[dst_v.at[B32]], add=True)

    plsc.subcore_barrier()
    _write_acc_slice(acc, out_hbm.at[c], base)


_DH2 = 64


@functools.partial(
    pl.kernel,
    out_type=jax.ShapeDtypeStruct((NC, NACC, _DH2), jnp.float32),
    mesh=_MESH,
    compiler_params=pltpu.CompilerParams(use_tc_tiling_on_sc=False),
    scratch_types=[
        pltpu.VMEM((B16 + 1, CHUNK), jnp.int32),
        pltpu.VMEM((B16 + 1, CHUNK), jnp.int32),
        [pltpu.VMEM((CHUNK, _DH2), jnp.float32) for _ in range(4)],
        [pltpu.SemaphoreType.DMA for _ in range(4)],
        [pltpu.SemaphoreType.DMA for _ in range(4)],
        pltpu.VMEM_SHARED((NACC, _DH2), jnp.float32),
    ],
)
def _msg128_kernel(edge_hbm, f0_hbm, f1_hbm, out_hbm,
                   src_v, dst_v, bufs, gsems, ssems, acc):
    c = lax.axis_index("c")
    s = lax.axis_index("s")
    base = s * 632

    _fill_2d(bufs[0], CHUNK, _DH2, 0.0)
    _zero_acc_slice(bufs[0], acc, base)
    plsc.subcore_barrier()

    _load_idx_16(edge_hbm, 0, s, src_v)
    _load_idx_16(edge_hbm, 1, s, dst_v)

    def run(feat_hbm):
        _msg_ring(feat_hbm, acc, src_v, dst_v, bufs, gsems, ssems, B16)

        @pl.when(s < X16)
        def _():
            _msg_tail(feat_hbm, acc, src_v, dst_v, bufs[0], gsems[0], B16)

    @pl.when(c == 0)
    def _():
        run(f0_hbm)

    @pl.when(c == 1)
    def _():
        run(f1_hbm)

    plsc.subcore_barrier()
    _write_acc_slice(acc, out_hbm.at[c], base)


_D2 = 16


@functools.partial(
    pl.kernel,
    out_type=jax.ShapeDtypeStruct((NC, NACC, _D2), jnp.float32),
    mesh=_MESH,
    compiler_params=pltpu.CompilerParams(use_tc_tiling_on_sc=False),
    scratch_types=[
        pltpu.VMEM((B32 + 1, CHUNK), jnp.int32),
        pltpu.VMEM((B32 + 1, CHUNK), jnp.int32),
        [pltpu.VMEM((CHUNK, _D2), jnp.float32) for _ in range(4)],
        [pltpu.SemaphoreType.DMA for _ in range(4)],
        [pltpu.SemaphoreType.DMA for _ in range(4)],
        pltpu.VMEM_SHARED((NACC, _D2), jnp.float32),
        pltpu.VMEM_SHARED((N, _D2), jnp.float32),
    ],
)
def _msg16_kernel(edge_hbm, feat_hbm, out_hbm,
                  src_v, dst_v, bufs, gsems, ssems, acc, feat_spm):
    c = lax.axis_index("c")
    s = lax.axis_index("s")
    wid = s * NC + c
    base = s * 632

    _fill_2d(bufs[0], CHUNK, _D2, 0.0)
    _zero_acc_slice(bufs[0], acc, base)
    for k in range(4):
        off = base + k * 128

        @pl.when(off + 128 <= N)
        def _(off=off):
            pltpu.sync_copy(feat_hbm.at[pl.ds(off, 128), :],
                            feat_spm.at[pl.ds(off, 128), :])

    @pl.when(base + 512 + 120 <= N)
    def _():
        pltpu.sync_copy(feat_hbm.at[pl.ds(base + 512, 120), :],
                        feat_spm.at[pl.ds(base + 512, 120), :])

    @pl.when(s == 15)
    def _():
        pltpu.sync_copy(feat_hbm.at[pl.ds(9984, 16), :],
                        feat_spm.at[pl.ds(9984, 16), :])

    plsc.subcore_barrier()

    _load_idx_32(edge_hbm, 0, wid, src_v)
    _load_idx_32(edge_hbm, 1, wid, dst_v)

    _msg_ring(feat_spm, acc, src_v, dst_v, bufs, gsems, ssems, 76)
    for j in (76, 77):
        _msg_tail(feat_spm, acc, src_v, dst_v, bufs[0], gsems[0], j)

    @pl.when(wid < X32)
    def _():
        _msg_tail(feat_spm, acc, src_v, dst_v, bufs[0], gsems[0], B32)

    plsc.subcore_barrier()
    _write_acc_slice(acc, out_hbm.at[c], base)


_RB = 2000
_GRID = N // _RB


def _dinv_of(degp_ref):
    deg = degp_ref[0, :, 0:1] + degp_ref[1, :, 0:1] + 1.0
    return lax.rsqrt(deg)


def _tc_pre_body(x_ref, w1_ref, degp_ref, f0_ref, f1_ref):
    dinv = _dinv_of(degp_ref)
    xw = dinv * jnp.dot(x_ref[...], w1_ref[...], preferred_element_type=jnp.float32)
    f0_ref[...] = xw[:, :_DH2]
    f1_ref[...] = xw[:, _DH2:]


def _tc_mid_body(a0_ref, a1_ref, f0_ref, f1_ref, degp_ref, b1_ref, w2_ref, out_ref):
    dinv = _dinv_of(degp_ref)
    h0 = jnp.tanh(dinv * (a0_ref[0] + f0_ref[...]) + b1_ref[:_DH2])
    h1 = jnp.tanh(dinv * (a1_ref[0] + f1_ref[...]) + b1_ref[_DH2:])
    hw2 = dinv * (
        jnp.dot(h0, w2_ref[:_DH2], preferred_element_type=jnp.float32)
        + jnp.dot(h1, w2_ref[_DH2:], preferred_element_type=jnp.float32))
    out_ref[...] = jnp.concatenate(
        [hw2, jnp.zeros((hw2.shape[0], _D2 - hw2.shape[1]), jnp.float32)], axis=1)


def _tc_post_body(c0_ref, c1_ref, hw_ref, degp_ref, b2_ref, wc_ref, bc_ref, out_ref):
    dinv = _dinv_of(degp_ref)
    nd = b2_ref.shape[0]
    emb = jnp.tanh(
        dinv * (c0_ref[0, :, :nd] + c1_ref[0, :, :nd] + hw_ref[:, :nd]) + b2_ref[...])
    out_ref[...] = jax.nn.sigmoid(
        jnp.dot(emb, wc_ref[...], preferred_element_type=jnp.float32) + bc_ref[...])


def _row_block(d):
    return pl.BlockSpec((_RB, d), lambda i: (i, 0))


def _acc_block(part, d):
    return pl.BlockSpec((1, _RB, d), lambda i, p=part: (p, i, 0))


def _deg_block():
    return pl.BlockSpec((2, _RB, DDEG), lambda i: (0, i, 0))


def _full(shape):
    return pl.BlockSpec(shape, lambda i: tuple(0 for _ in shape))


def kernel(x, edge_index, W1, b1, W2, b2, Wc, bc):
    f32 = jnp.float32
    edge3d = edge_index.reshape(2, NCH, CHUNK)

    degp = _deg_kernel(edge3d)

    f0, f1 = pl.pallas_call(
        _tc_pre_body,
        grid=(_GRID,),
        in_specs=[_row_block(D_IN), _full((D_IN, D_H)), _deg_block()],
        out_specs=[_row_block(_DH2), _row_block(_DH2)],
        out_shape=[jax.ShapeDtypeStruct((N, _DH2), f32),
                   jax.ShapeDtypeStruct((N, _DH2), f32)],
    )(x, W1, degp)

    acc1 = _msg128_kernel(edge3d, f0, f1)

    hw2s = pl.pallas_call(
        _tc_mid_body,
        grid=(_GRID,),
        in_specs=[_acc_block(0, _DH2), _acc_block(1, _DH2),
                  _row_block(_DH2), _row_block(_DH2),
                  _deg_block(), _full((D_H,)), _full((D_H, W2.shape[1]))],
        out_specs=_row_block(_D2),
        out_shape=jax.ShapeDtypeStruct((N, _D2), f32),
    )(acc1, acc1, f0, f1, degp, b1, W2)

    acc2 = _msg16_kernel(edge3d, hw2s)

    out = pl.pallas_call(
        _tc_post_body,
        grid=(_GRID,),
        in_specs=[_acc_block(0, _D2), _acc_block(1, _D2), _row_block(_D2),
                  _deg_block(), _full((W2.shape[1],)),
                  _full((Wc.shape[0], Wc.shape[1])), _full((Wc.shape[1],))],
        out_specs=_row_block(Wc.shape[1]),
        out_shape=jax.ShapeDtypeStruct((N, Wc.shape[1]), f32),
    )(acc2, acc2, hw2s, degp, b2, Wc, bc)

    return out
